# baseline (speedup 1.0000x reference)
import jax
import jax.numpy as jnp
from jax import lax
from jax.experimental import pallas as pl
from jax.experimental.pallas import tpu as pltpu

K = 10
N = 4_000_000
BL = 160000
NB = N // BL


def _qwk_from_cm(cm):
    cm = cm / jnp.float32(N)
    mt = jnp.sum(cm, axis=1, keepdims=True)
    mp = jnp.sum(cm, axis=0, keepdims=True)
    expected = mt * mp
    i = lax.broadcasted_iota(jnp.int32, (K, K), 0).astype(jnp.float32)
    j = lax.broadcasted_iota(jnp.int32, (K, K), 1).astype(jnp.float32)
    w = 1.0 - (i - j) ** 2 / float((K - 1) ** 2)
    eps = 1e-07
    po = jnp.sum(w * cm)
    pe = jnp.clip(jnp.sum(w * expected), 0.0, 1.0 - eps)
    qwk = jnp.clip((po - pe) / (1.0 - pe + eps), -1.0, 1.0)
    return jnp.reshape(1.0 - qwk, (1, 1))


def _cm_body(xT_ref, tgt_ref, o_ref, acc_ref):
    b = pl.program_id(0)
    x = xT_ref[...]
    m = jnp.max(x, axis=0, keepdims=True)
    sub = lax.broadcasted_iota(jnp.int32, (K, BL), 0)
    ponehot = (x == m).astype(jnp.bfloat16)
    tonehot = (sub == tgt_ref[0]).astype(jnp.bfloat16)
    partial = lax.dot_general(tonehot, ponehot,
                              (((1,), (1,)), ((), ())),
                              preferred_element_type=jnp.float32)

    @pl.when(b == 0)
    def _():
        acc_ref[...] = jnp.zeros((K, K), jnp.float32)

    acc_ref[...] += partial

    @pl.when(b == NB - 1)
    def _():
        o_ref[...] = acc_ref[...]


def _qwk_body(cm_ref, o_ref):
    o_ref[...] = _qwk_from_cm(cm_ref[...])


@jax.jit
def kernel(logits, targets):
    xT = jnp.transpose(logits)
    tgt3 = targets.reshape(NB, 1, BL)
    cm = pl.pallas_call(
        _cm_body,
        grid=(NB,),
        in_specs=[
            pl.BlockSpec((K, BL), lambda i: (0, i)),
            pl.BlockSpec((1, 1, BL), lambda i: (i, 0, 0)),
        ],
        out_specs=pl.BlockSpec((K, K), lambda i: (0, 0)),
        out_shape=jax.ShapeDtypeStruct((K, K), jnp.float32),
        scratch_shapes=[pltpu.VMEM((K, K), jnp.float32)],
    )(xT, tgt3)
    out = pl.pallas_call(
        _qwk_body,
        out_shape=jax.ShapeDtypeStruct((1, 1), jnp.float32),
    )(cm)
    return out.reshape(())
